# baseline (speedup 1.0000x reference)
import functools

import jax
import jax.numpy as jnp
from jax import lax
from jax.experimental import pallas as pl
from jax.experimental.pallas import tpu as pltpu
from jax.experimental.pallas import tpu_sc as plsc

_PLANES = [32, 64, 128, 256, 512]
_BLOCKS = [2, 3, 4, 6, 3]
_STRIDE = [1, 4, 4, 4, 4]
_NSAMPLE = [8, 16, 16, 16, 16]
_B = 2
_N = 4096

_INTERPRET = False


def _pad_cols(a, d):
    if a.shape[1] == d:
        return a
    return jnp.concatenate(
        [a, jnp.zeros((a.shape[0], d - a.shape[1]), a.dtype)], axis=1)


def _sc_gather(table, idx):
    R, D = table.shape
    Bn = idx.shape[0]
    NW = 32
    assert Bn % (8 * NW) == 0 and D % 16 == 0, (Bn, D)
    bpw = Bn // NW
    chunk = bpw
    while chunk * D > 98304:
        chunk //= 2
    nchunks = bpw // chunk
    mesh = plsc.VectorSubcoreMesh(core_axis_name="c", subcore_axis_name="s")

    @functools.partial(
        pl.kernel,
        out_type=jax.ShapeDtypeStruct((Bn, D), jnp.float32),
        mesh=mesh,
        compiler_params=pltpu.CompilerParams(use_tc_tiling_on_sc=False),
        scratch_types=[
            pltpu.VMEM((chunk,), jnp.int32),
            pltpu.VMEM((chunk, D), jnp.float32),
            pltpu.SemaphoreType.DMA,
        ],
    )
    def gk(table_hbm, idx_hbm, out_hbm, idx_v, rows_v, sem):
        wid = lax.axis_index("s") * 2 + lax.axis_index("c")
        base = wid * bpw
        for t in range(nchunks):
            off = base + t * chunk
            pltpu.sync_copy(idx_hbm.at[pl.ds(off, chunk)], idx_v)
            pltpu.async_copy(table_hbm.at[idx_v], rows_v, sem).wait()
            pltpu.sync_copy(rows_v, out_hbm.at[pl.ds(off, chunk)])

    return gk(table, idx)


def _knn_pl(q, pT8, k):
    Bb, M, _ = q.shape
    n = pT8.shape[2]
    bm = min(M, 256)

    def body(q_ref, p_ref, o_ref):
        q3 = q_ref[0]
        d = None
        for t in range(3):
            dt = q3[:, t:t + 1] - p_ref[0, t:t + 1, :]
            dt = dt * dt
            d = dt if d is None else d + dt
        iota = lax.broadcasted_iota(jnp.int32, (bm, n), 1)
        cols = []
        for _ in range(k):
            mv = jnp.min(d, axis=1, keepdims=True)
            idx = jnp.min(jnp.where(d == mv, iota, n), axis=1, keepdims=True)
            cols.append(idx)
            d = jnp.where(iota == idx, jnp.inf, d)
        o_ref[0] = jnp.concatenate(cols, axis=1)

    return pl.pallas_call(
        body,
        grid=(Bb, M // bm),
        in_specs=[
            pl.BlockSpec((1, bm, 3), lambda b, i: (b, i, 0)),
            pl.BlockSpec((1, 8, n), lambda b, i: (b, 0, 0)),
        ],
        out_specs=pl.BlockSpec((1, bm, k), lambda b, i: (b, i, 0)),
        out_shape=jax.ShapeDtypeStruct((Bb, M, k), jnp.int32),
        interpret=_INTERPRET,
    )(q, pT8)


def _fps_pl(p4, m):
    Bb = p4.shape[0]
    n8 = p4.shape[3]
    n = 8 * n8

    def body(p_ref, o_ref):
        px = p_ref[0, 0]
        py = p_ref[0, 1]
        pz = p_ref[0, 2]
        iota = (lax.broadcasted_iota(jnp.int32, (8, n8), 0) * n8
                + lax.broadcasted_iota(jnp.int32, (8, n8), 1))
        iom = lax.broadcasted_iota(jnp.int32, (1, m), 1)

        def red2(a, fn):
            return fn(fn(a, axis=1, keepdims=True), axis=0, keepdims=True)

        def step(i, carry):
            dists, rx, ry, rz, cx, cy, cz = carry
            dx = px - cx
            dy = py - cy
            dz = pz - cz
            d = dx * dx + dy * dy + dz * dz
            dists = jnp.minimum(dists, d)
            mv = red2(dists, jnp.max)
            idx = red2(jnp.where(dists == mv, iota, n), jnp.min)
            sel = iota == idx
            cx = red2(jnp.where(sel, px, 0.0), jnp.sum)
            cy = red2(jnp.where(sel, py, 0.0), jnp.sum)
            cz = red2(jnp.where(sel, pz, 0.0), jnp.sum)
            rx = jnp.where(iom == i, cx, rx)
            ry = jnp.where(iom == i, cy, ry)
            rz = jnp.where(iom == i, cz, rz)
            return dists, rx, ry, rz, cx, cy, cz

        c0x = px[0:1, 0:1]
        c0y = py[0:1, 0:1]
        c0z = pz[0:1, 0:1]
        zrow = jnp.zeros((1, m), jnp.float32)
        init = (jnp.full((8, n8), 1e10, jnp.float32),
                jnp.where(iom == 0, c0x, zrow),
                jnp.where(iom == 0, c0y, zrow),
                jnp.where(iom == 0, c0z, zrow),
                c0x, c0y, c0z)
        _, rx, ry, rz, _, _, _ = lax.fori_loop(1, m, step, init)
        o_ref[0, 0:1, :] = rx
        o_ref[0, 1:2, :] = ry
        o_ref[0, 2:3, :] = rz
        o_ref[0, 3:8, :] = jnp.zeros((5, m), jnp.float32)

    return pl.pallas_call(
        body,
        grid=(Bb,),
        in_specs=[pl.BlockSpec((1, 3, 8, n8), lambda b: (b, 0, 0, 0))],
        out_specs=pl.BlockSpec((1, 8, m), lambda b: (b, 0, 0)),
        out_shape=jax.ShapeDtypeStruct((Bb, 8, m), jnp.float32),
        interpret=_INTERPRET,
    )(p4)


def _lin_relu_pl(x, w, g, b):
    M, cin = x.shape
    c = w.shape[1]
    bm = min(M, 1024)

    def body(x_ref, w_ref, g_ref, b_ref, o_ref):
        h = jnp.dot(x_ref[...], w_ref[...], preferred_element_type=jnp.float32)
        o_ref[...] = jnp.maximum(h * g_ref[...] + b_ref[...], 0.0)

    return pl.pallas_call(
        body,
        grid=(M // bm,),
        in_specs=[
            pl.BlockSpec((bm, cin), lambda i: (i, 0)),
            pl.BlockSpec((cin, c), lambda i: (0, 0)),
            pl.BlockSpec((1, c), lambda i: (0, 0)),
            pl.BlockSpec((1, c), lambda i: (0, 0)),
        ],
        out_specs=pl.BlockSpec((bm, c), lambda i: (i, 0)),
        out_shape=jax.ShapeDtypeStruct((M, c), jnp.float32),
        interpret=_INTERPRET,
    )(x, w, g[None, :], b[None, :])


def _td_pl(gpx, npos, wpad, g, b, ns, c):
    M = npos.shape[0]
    Dp = gpx.shape[1]
    bm = min(M, 256)

    def body(gpx_ref, np_ref, w_ref, g_ref, b_ref, o_ref):
        wfull = w_ref[...]
        h = jnp.dot(gpx_ref[...], wfull, preferred_element_type=jnp.float32)
        corr = jnp.dot(np_ref[...], wfull[:3], preferred_element_type=jnp.float32)
        h3 = h.reshape(bm, ns, c) - corr[:, None, :]
        h3 = jnp.maximum(h3 * g_ref[...][None] + b_ref[...][None], 0.0)
        out = h3[:, 0, :]
        for j in range(1, ns):
            out = jnp.maximum(out, h3[:, j, :])
        o_ref[...] = out

    return pl.pallas_call(
        body,
        grid=(M // bm,),
        in_specs=[
            pl.BlockSpec((bm * ns, Dp), lambda i: (i, 0)),
            pl.BlockSpec((bm, 3), lambda i: (i, 0)),
            pl.BlockSpec((Dp, c), lambda i: (0, 0)),
            pl.BlockSpec((1, c), lambda i: (0, 0)),
            pl.BlockSpec((1, c), lambda i: (0, 0)),
        ],
        out_specs=pl.BlockSpec((bm, c), lambda i: (i, 0)),
        out_shape=jax.ShapeDtypeStruct((M, c), jnp.float32),
        interpret=_INTERPRET,
    )(gpx, npos, wpad, g[None, :], b[None, :])


def _td_oh_pl(ptabx, kidx, npos, wpad, g, b, ns, c):
    nt, Dp = ptabx.shape
    M = npos.shape[0]

    def body(tab_ref, idx_ref, np_ref, w_ref, g_ref, b_ref, o_ref):
        MN = M * ns
        idx3 = idx_ref[...][:, :, None]
        iota = lax.broadcasted_iota(jnp.int32, (M, ns, nt), 2)
        oh = jnp.where(iota == idx3, 1.0, 0.0).reshape(MN, nt)
        gpx = jnp.dot(oh, tab_ref[...], preferred_element_type=jnp.float32)
        wfull = w_ref[...]
        h = jnp.dot(gpx, wfull, preferred_element_type=jnp.float32)
        corr = jnp.dot(np_ref[...], wfull[:3], preferred_element_type=jnp.float32)
        h3 = h.reshape(M, ns, c) - corr[:, None, :]
        h3 = jnp.maximum(h3 * g_ref[...][None] + b_ref[...][None], 0.0)
        out = h3[:, 0, :]
        for j in range(1, ns):
            out = jnp.maximum(out, h3[:, j, :])
        o_ref[...] = out

    return pl.pallas_call(
        body,
        out_shape=jax.ShapeDtypeStruct((M, c), jnp.float32),
        interpret=_INTERPRET,
    )(ptabx, kidx, npos, wpad, g[None, :], b[None, :])


def _blocks_fused_pl(x, p16, bidx, blks, ns, c):
    M = x.shape[0]
    cs = c // 8
    nb = len(blks)

    def body(*refs):
        x_ref, p16_ref, bidx_ref = refs[0:3]
        prefs = refs[3:-1]
        o_ref = refs[-1]
        MN = M * ns
        idx3 = bidx_ref[...][:, :, None]
        iota = lax.broadcasted_iota(jnp.int32, (M, ns, M), 2)
        oh = jnp.where(iota == idx3, 1.0, 0.0).reshape(MN, M)
        p16full = p16_ref[...]
        pg16 = jnp.dot(oh, p16full, preferred_element_type=jnp.float32)
        xcur = x_ref[...]
        for bi in range(nb):
            (w1, g1, b1, wq, bq, wk, bk, wv, bv,
             wp1, bp1, gp, bpn, wp2, bp2,
             gw1, bw1, ww1, bww1, gw2, bw2, ww2, bww2,
             g2, b2, w3, g3, b3) = [r[...] for r in prefs[bi * 28:(bi + 1) * 28]]
            y = jnp.maximum(
                jnp.dot(xcur, w1, preferred_element_type=jnp.float32) * g1 + b1,
                0.0)
            q = jnp.dot(y, wq, preferred_element_type=jnp.float32) + bq
            kk = jnp.dot(y, wk, preferred_element_type=jnp.float32) + bk
            vv = jnp.dot(y, wv, preferred_element_type=jnp.float32) + bv
            kv = jnp.concatenate([kk, vv], axis=1)
            gkv = jnp.dot(oh, kv, preferred_element_type=jnp.float32)
            a = jnp.dot(pg16, wp1, preferred_element_type=jnp.float32)
            corr = jnp.dot(p16full[:, :3], wp1[:3],
                           preferred_element_type=jnp.float32)
            a3 = a.reshape(M, ns, 3) - corr[:, None, :] + bp1[None]
            a3 = jnp.maximum(a3 * gp[None] + bpn[None], 0.0)
            pr = jnp.dot(a3.reshape(MN, 3), wp2,
                         preferred_element_type=jnp.float32) + bp2
            gk = gkv[:, :c]
            gv = gkv[:, c:]
            t3 = (gk + pr).reshape(M, ns, c) - q[:, None, :]
            w = jnp.maximum(t3 * gw1[None] + bw1[None], 0.0)
            w = jnp.dot(w.reshape(MN, c), ww1,
                        preferred_element_type=jnp.float32) + bww1
            w = jnp.maximum(w * gw2 + bw2, 0.0)
            w = jnp.dot(w, ww2, preferred_element_type=jnp.float32) + bww2
            w3d = w.reshape(M, ns, cs)
            mx = w3d[:, 0:1, :]
            for j in range(1, ns):
                mx = jnp.maximum(mx, w3d[:, j:j + 1, :])
            e = jnp.exp(w3d - mx)
            ssum = e[:, 0:1, :]
            for j in range(1, ns):
                ssum = ssum + e[:, j:j + 1, :]
            sm = e / ssum
            wfull = jnp.concatenate([sm] * 8, axis=2)
            vpr = (gv + pr).reshape(M, ns, c) * wfull
            out = vpr[:, 0, :]
            for j in range(1, ns):
                out = out + vpr[:, j, :]
            z = jnp.maximum(out * g2 + b2, 0.0)
            o = (jnp.dot(z, w3, preferred_element_type=jnp.float32)
                 * g3 + b3 + xcur)
            xcur = jnp.maximum(o, 0.0)
        o_ref[...] = xcur

    ops = [x, p16, bidx]
    for bp in blks:
        ap = bp['attn']
        wp1pad = jnp.zeros((16, 3), jnp.float32).at[:3].set(ap['wp1'])
        ops += [bp['w1'], bp['g1'][None, :], bp['b1'][None, :],
                ap['wq'], ap['bq'][None, :], ap['wk'], ap['bk'][None, :],
                ap['wv'], ap['bv'][None, :],
                wp1pad, ap['bp1'][None, :], ap['gp'][None, :],
                ap['bpn'][None, :], ap['wp2'], ap['bp2'][None, :],
                ap['gw1'][None, :], ap['bw1'][None, :], ap['ww1'],
                ap['bww1'][None, :], ap['gw2'][None, :], ap['bw2'][None, :],
                ap['ww2'], ap['bww2'][None, :],
                bp['g2'][None, :], bp['b2'][None, :], bp['w3'],
                bp['g3'][None, :], bp['b3'][None, :]]
    return pl.pallas_call(
        body,
        out_shape=jax.ShapeDtypeStruct((M, c), jnp.float32),
        interpret=_INTERPRET,
    )(*ops)


def _qkv_pl(x, w1, g1, b1, wq, bq, wk, bk, wv, bv):
    M, c = x.shape
    bm = min(M, 1024)

    def body(x_ref, w1_r, g1_r, b1_r, wq_r, bq_r, wk_r, bk_r, wv_r, bv_r,
             q_ref, kv_ref):
        xb = x_ref[...]
        y = jnp.maximum(
            jnp.dot(xb, w1_r[...], preferred_element_type=jnp.float32)
            * g1_r[...] + b1_r[...], 0.0)
        q_ref[...] = jnp.dot(y, wq_r[...], preferred_element_type=jnp.float32) + bq_r[...]
        kk = jnp.dot(y, wk_r[...], preferred_element_type=jnp.float32) + bk_r[...]
        vv = jnp.dot(y, wv_r[...], preferred_element_type=jnp.float32) + bv_r[...]
        kv_ref[...] = jnp.concatenate([kk, vv], axis=1)

    mat = lambda: pl.BlockSpec((c, c), lambda i: (0, 0))
    vec = lambda: pl.BlockSpec((1, c), lambda i: (0, 0))
    return pl.pallas_call(
        body,
        grid=(M // bm,),
        in_specs=[pl.BlockSpec((bm, c), lambda i: (i, 0)),
                  mat(), vec(), vec(), mat(), vec(), mat(), vec(), mat(), vec()],
        out_specs=[pl.BlockSpec((bm, c), lambda i: (i, 0)),
                   pl.BlockSpec((bm, 2 * c), lambda i: (i, 0))],
        out_shape=[jax.ShapeDtypeStruct((M, c), jnp.float32),
                   jax.ShapeDtypeStruct((M, 2 * c), jnp.float32)],
        interpret=_INTERPRET,
    )(x, w1, g1[None, :], b1[None, :], wq, bq[None, :], wk, bk[None, :],
      wv, bv[None, :])


def _attn_pl(xin, q, gkv, pg, p, ns, c, ap, g2, b2, w3m, g3, b3):
    M = xin.shape[0]
    cs = c // 8
    bm = min(M, 256)
    wp1pad = jnp.zeros((16, 3), jnp.float32).at[:3].set(ap['wp1'])

    def body(xin_ref, q_ref, gkv_ref, pg_ref, p_ref,
             wp1_r, bp1_r, gp_r, bpn_r, wp2_r, bp2_r,
             gw1_r, bw1_r, ww1_r, bww1_r, gw2_r, bw2_r, ww2_r, bww2_r,
             g2_r, b2_r, w3_r, g3_r, b3_r, o_ref):
        BN = bm * ns
        a = jnp.dot(pg_ref[...], wp1_r[...], preferred_element_type=jnp.float32)
        corr = jnp.dot(p_ref[...], wp1_r[...][:3],
                       preferred_element_type=jnp.float32)
        a3 = a.reshape(bm, ns, 3) - corr[:, None, :] + bp1_r[...][None]
        a3 = jnp.maximum(a3 * gp_r[...][None] + bpn_r[...][None], 0.0)
        pr = jnp.dot(a3.reshape(BN, 3), wp2_r[...],
                     preferred_element_type=jnp.float32) + bp2_r[...]
        g = gkv_ref[...]
        gk = g[:, :c]
        gv = g[:, c:]
        t3 = (gk + pr).reshape(bm, ns, c) - q_ref[...][:, None, :]
        w = jnp.maximum(t3 * gw1_r[...][None] + bw1_r[...][None], 0.0)
        w = jnp.dot(w.reshape(BN, c), ww1_r[...],
                    preferred_element_type=jnp.float32) + bww1_r[...]
        w = jnp.maximum(w * gw2_r[...] + bw2_r[...], 0.0)
        w = jnp.dot(w, ww2_r[...], preferred_element_type=jnp.float32) + bww2_r[...]
        w3d = w.reshape(bm, ns, cs)
        mx = w3d[:, 0:1, :]
        for j in range(1, ns):
            mx = jnp.maximum(mx, w3d[:, j:j + 1, :])
        e = jnp.exp(w3d - mx)
        ssum = e[:, 0:1, :]
        for j in range(1, ns):
            ssum = ssum + e[:, j:j + 1, :]
        sm = e / ssum
        wfull = jnp.concatenate([sm] * 8, axis=2)
        vpr = (gv + pr).reshape(bm, ns, c) * wfull
        out = vpr[:, 0, :]
        for j in range(1, ns):
            out = out + vpr[:, j, :]
        z = jnp.maximum(out * g2_r[...] + b2_r[...], 0.0)
        o = (jnp.dot(z, w3_r[...], preferred_element_type=jnp.float32)
             * g3_r[...] + b3_r[...] + xin_ref[...])
        o_ref[...] = jnp.maximum(o, 0.0)

    row = lambda w: pl.BlockSpec((bm, w), lambda i: (i, 0))
    rowns = lambda w: pl.BlockSpec((bm * ns, w), lambda i: (i, 0))
    full = lambda a, bdim: pl.BlockSpec((a, bdim), lambda i: (0, 0))
    cs_ = cs
    return pl.pallas_call(
        body,
        grid=(M // bm,),
        in_specs=[
            row(c), row(c), rowns(2 * c), rowns(16), row(3),
            full(16, 3), full(1, 3), full(1, 3), full(1, 3),
            full(3, c), full(1, c),
            full(1, c), full(1, c), full(c, cs_), full(1, cs_),
            full(1, cs_), full(1, cs_), full(cs_, cs_), full(1, cs_),
            full(1, c), full(1, c), full(c, c), full(1, c), full(1, c),
        ],
        out_specs=row(c),
        out_shape=jax.ShapeDtypeStruct((M, c), jnp.float32),
        interpret=_INTERPRET,
    )(xin, q, gkv, pg, p,
      wp1pad, ap['bp1'][None, :], ap['gp'][None, :], ap['bpn'][None, :],
      ap['wp2'], ap['bp2'][None, :],
      ap['gw1'][None, :], ap['bw1'][None, :], ap['ww1'], ap['bww1'][None, :],
      ap['gw2'][None, :], ap['bw2'][None, :], ap['ww2'], ap['bww2'][None, :],
      g2[None, :], b2[None, :], w3m, g3[None, :], b3[None, :])


def _pT8(p):
    Bb, n, _ = p.shape
    pt = jnp.transpose(p, (0, 2, 1))
    return jnp.concatenate([pt, jnp.zeros((Bb, 5, n), jnp.float32)], axis=1)


def kernel(inputs, params):
    pxo = jnp.transpose(inputs[0], (0, 2, 1))
    p = pxo[:, :, :3]
    x = pxo
    n = _N
    outs = []
    for i in range(5):
        sp = params[i]
        ns = _NSAMPLE[i]
        c = _PLANES[i]
        td = sp['td']
        if _STRIDE[i] == 1:
            x = _lin_relu_pl(x.reshape(_B * n, -1), td['w'], td['g'],
                             td['b']).reshape(_B, n, c)
        else:
            cin = x.shape[-1]
            m = n // _STRIDE[i]
            p4 = jnp.transpose(p, (0, 2, 1)).reshape(_B, 3, 8, n // 8)
            nposT = _fps_pl(p4, m)
            nposb = jnp.transpose(nposT[:, :3, :], (0, 2, 1))
            npos = nposb.reshape(_B * m, 3)
            goff = (jnp.arange(_B, dtype=jnp.int32) * n)[:, None]
            kidx = _knn_pl(nposb, _pT8(p), ns)
            kglob = (kidx + goff[:, :, None]).reshape(_B * m, ns)
            Dp = ((8 + cin + 15) // 16) * 16
            ptabx = _pad_cols(
                jnp.concatenate([_pad_cols(p.reshape(_B * n, 3), 8),
                                 x.reshape(_B * n, cin)], axis=1), Dp)
            wpad = jnp.zeros((Dp, c), jnp.float32)
            wpad = wpad.at[:3].set(td['w'][:3]).at[8:8 + cin].set(td['w'][3:])
            if _B * n <= 512:
                x = _td_oh_pl(ptabx, kglob, npos, wpad, td['g'], td['b'],
                              ns, c)
            else:
                gpx = _sc_gather(ptabx, kglob.reshape(-1))
                x = _td_pl(gpx, npos, wpad, td['g'], td['b'], ns, c)
            x = x.reshape(_B, m, c)
            p = nposb
            n = m
        goff = (jnp.arange(_B, dtype=jnp.int32) * n)[:, None, None]
        bidx = _knn_pl(p, _pT8(p), ns)
        bglob = (bidx + goff).reshape(_B * n, ns)
        if _B * n <= 128:
            p16 = _pad_cols(p.reshape(_B * n, 3), 16)
            xf = _blocks_fused_pl(x.reshape(_B * n, c), p16, bglob,
                                  sp['blocks'], ns, c)
            x = xf.reshape(_B, n, c)
        else:
            bflat = bglob.reshape(-1)
            ptab = _pad_cols(p.reshape(_B * n, 3), 16)
            pg = _sc_gather(ptab, bflat)
            pflat = p.reshape(_B * n, 3)
            for bp in sp['blocks']:
                xf = x.reshape(_B * n, c)
                ap = bp['attn']
                q, kv = _qkv_pl(xf, bp['w1'], bp['g1'], bp['b1'],
                                ap['wq'], ap['bq'], ap['wk'], ap['bk'],
                                ap['wv'], ap['bv'])
                gkv = _sc_gather(kv, bflat)
                xf = _attn_pl(xf, q, gkv, pg, pflat, ns, c, ap,
                              bp['g2'], bp['b2'], bp['w3'], bp['g3'],
                              bp['b3'])
                x = xf.reshape(_B, n, c)
        outs.append((p, x))
    res = []
    for pp, xx in outs:
        res.append(pp.reshape(-1, 3))
        res.append(xx.reshape(-1, xx.shape[-1]))
    return tuple(res)


# speedup vs baseline: 1.3527x; 1.3527x over previous
import functools

import jax
import jax.numpy as jnp
from jax import lax
from jax.experimental import pallas as pl
from jax.experimental.pallas import tpu as pltpu
from jax.experimental.pallas import tpu_sc as plsc

_PLANES = [32, 64, 128, 256, 512]
_BLOCKS = [2, 3, 4, 6, 3]
_STRIDE = [1, 4, 4, 4, 4]
_NSAMPLE = [8, 16, 16, 16, 16]
_B = 2
_N = 4096

_INTERPRET = False


def _pad_cols(a, d):
    if a.shape[1] == d:
        return a
    return jnp.concatenate(
        [a, jnp.zeros((a.shape[0], d - a.shape[1]), a.dtype)], axis=1)


def _sc_gather(table, idx):
    R, D = table.shape
    Bn = idx.shape[0]
    NW = 32
    assert Bn % (8 * NW) == 0 and D % 16 == 0, (Bn, D)
    bpw = Bn // NW
    chunk = bpw
    while chunk * D > 98304:
        chunk //= 2
    nchunks = bpw // chunk
    mesh = plsc.VectorSubcoreMesh(core_axis_name="c", subcore_axis_name="s")

    @functools.partial(
        pl.kernel,
        out_type=jax.ShapeDtypeStruct((Bn, D), jnp.float32),
        mesh=mesh,
        compiler_params=pltpu.CompilerParams(use_tc_tiling_on_sc=False),
        scratch_types=[
            pltpu.VMEM((chunk,), jnp.int32),
            pltpu.VMEM((chunk, D), jnp.float32),
            pltpu.SemaphoreType.DMA,
        ],
    )
    def gk(table_hbm, idx_hbm, out_hbm, idx_v, rows_v, sem):
        wid = lax.axis_index("s") * 2 + lax.axis_index("c")
        base = wid * bpw
        for t in range(nchunks):
            off = base + t * chunk
            pltpu.sync_copy(idx_hbm.at[pl.ds(off, chunk)], idx_v)
            pltpu.async_copy(table_hbm.at[idx_v], rows_v, sem).wait()
            pltpu.sync_copy(rows_v, out_hbm.at[pl.ds(off, chunk)])

    return gk(table, idx)


def _knn_pl(q, pT8, k):
    Bb, M, _ = q.shape
    n = pT8.shape[2]
    bm = min(M, 256)

    def body(q_ref, p_ref, o_ref):
        q3 = q_ref[0]
        d = None
        for t in range(3):
            dt = q3[:, t:t + 1] - p_ref[0, t:t + 1, :]
            dt = dt * dt
            d = dt if d is None else d + dt
        iota = lax.broadcasted_iota(jnp.int32, (bm, n), 1)
        cols = []
        for _ in range(k):
            mv = jnp.min(d, axis=1, keepdims=True)
            idx = jnp.min(jnp.where(d == mv, iota, n), axis=1, keepdims=True)
            cols.append(idx)
            d = jnp.where(iota == idx, jnp.inf, d)
        o_ref[0] = jnp.concatenate(cols, axis=1)

    return pl.pallas_call(
        body,
        grid=(Bb, M // bm),
        in_specs=[
            pl.BlockSpec((1, bm, 3), lambda b, i: (b, i, 0)),
            pl.BlockSpec((1, 8, n), lambda b, i: (b, 0, 0)),
        ],
        out_specs=pl.BlockSpec((1, bm, k), lambda b, i: (b, i, 0)),
        out_shape=jax.ShapeDtypeStruct((Bb, M, k), jnp.int32),
        interpret=_INTERPRET,
    )(q, pT8)


def _fps_pl(p4, m):
    Bb = p4.shape[0]
    n8 = p4.shape[3]
    n = 8 * n8
    R = Bb * 8

    def body(p_ref, o_ref):
        px = p_ref[:, 0].reshape(R, n8)
        py = p_ref[:, 1].reshape(R, n8)
        pz = p_ref[:, 2].reshape(R, n8)
        P = jnp.concatenate([px, py, pz], axis=0)
        iota = (lax.broadcasted_iota(jnp.int32, (R, n8), 0) % 8 * n8
                + lax.broadcasted_iota(jnp.int32, (R, n8), 1))
        iom = lax.broadcasted_iota(jnp.int32, (1, m), 1)

        def redb(a, fn, nb):
            r = fn(a, axis=1, keepdims=True).reshape(nb, 8, 1)
            return fn(r, axis=1, keepdims=True)

        def bexp(a):
            return jnp.broadcast_to(a, (Bb, 8, n8)).reshape(R, n8)

        def step(i, carry):
            dists, rx, ry, rz, cb = carry
            dx = px - bexp(cb[0:Bb])
            dy = py - bexp(cb[Bb:2 * Bb])
            dz = pz - bexp(cb[2 * Bb:])
            d = dx * dx + dy * dy + dz * dz
            dists = jnp.minimum(dists, d)
            mv = redb(dists, jnp.max, Bb)
            idx = redb(jnp.where(dists == bexp(mv), iota, n), jnp.min, Bb)
            sel = iota == bexp(idx)
            sel3 = jnp.concatenate([sel, sel, sel], axis=0)
            cb = redb(jnp.where(sel3, P, 0.0), jnp.sum, 3 * Bb)
            rx = jnp.where(iom == i, cb[0:Bb, :, 0], rx)
            ry = jnp.where(iom == i, cb[Bb:2 * Bb, :, 0], ry)
            rz = jnp.where(iom == i, cb[2 * Bb:, :, 0], rz)
            return dists, rx, ry, rz, cb

        sel00 = iota == 0
        sel003 = jnp.concatenate([sel00, sel00, sel00], axis=0)
        c0 = redb(jnp.where(sel003, P, 0.0), jnp.sum, 3 * Bb)
        zrow = jnp.zeros((Bb, m), jnp.float32)
        init = (jnp.full((R, n8), 1e10, jnp.float32),
                jnp.where(iom == 0, c0[0:Bb, :, 0], zrow),
                jnp.where(iom == 0, c0[Bb:2 * Bb, :, 0], zrow),
                jnp.where(iom == 0, c0[2 * Bb:, :, 0], zrow),
                c0)
        _, rx, ry, rz, _ = lax.fori_loop(1, m, step, init)
        o_ref[:, 0:1, :] = rx[:, None, :]
        o_ref[:, 1:2, :] = ry[:, None, :]
        o_ref[:, 2:3, :] = rz[:, None, :]
        o_ref[:, 3:8, :] = jnp.zeros((Bb, 5, m), jnp.float32)

    return pl.pallas_call(
        body,
        out_shape=jax.ShapeDtypeStruct((Bb, 8, m), jnp.float32),
        interpret=_INTERPRET,
    )(p4)


def _lin_relu_pl(x, w, g, b):
    M, cin = x.shape
    c = w.shape[1]
    bm = min(M, 1024)

    def body(x_ref, w_ref, g_ref, b_ref, o_ref):
        h = jnp.dot(x_ref[...], w_ref[...], preferred_element_type=jnp.float32)
        o_ref[...] = jnp.maximum(h * g_ref[...] + b_ref[...], 0.0)

    return pl.pallas_call(
        body,
        grid=(M // bm,),
        in_specs=[
            pl.BlockSpec((bm, cin), lambda i: (i, 0)),
            pl.BlockSpec((cin, c), lambda i: (0, 0)),
            pl.BlockSpec((1, c), lambda i: (0, 0)),
            pl.BlockSpec((1, c), lambda i: (0, 0)),
        ],
        out_specs=pl.BlockSpec((bm, c), lambda i: (i, 0)),
        out_shape=jax.ShapeDtypeStruct((M, c), jnp.float32),
        interpret=_INTERPRET,
    )(x, w, g[None, :], b[None, :])


def _td_pl(gpx, npos, wpad, g, b, ns, c):
    M = npos.shape[0]
    Dp = gpx.shape[1]
    bm = min(M, 256)

    def body(gpx_ref, np_ref, w_ref, g_ref, b_ref, o_ref):
        wfull = w_ref[...]
        h = jnp.dot(gpx_ref[...], wfull, preferred_element_type=jnp.float32)
        corr = jnp.dot(np_ref[...], wfull[:3], preferred_element_type=jnp.float32)
        h3 = h.reshape(bm, ns, c) - corr[:, None, :]
        h3 = jnp.maximum(h3 * g_ref[...][None] + b_ref[...][None], 0.0)
        out = h3[:, 0, :]
        for j in range(1, ns):
            out = jnp.maximum(out, h3[:, j, :])
        o_ref[...] = out

    return pl.pallas_call(
        body,
        grid=(M // bm,),
        in_specs=[
            pl.BlockSpec((bm * ns, Dp), lambda i: (i, 0)),
            pl.BlockSpec((bm, 3), lambda i: (i, 0)),
            pl.BlockSpec((Dp, c), lambda i: (0, 0)),
            pl.BlockSpec((1, c), lambda i: (0, 0)),
            pl.BlockSpec((1, c), lambda i: (0, 0)),
        ],
        out_specs=pl.BlockSpec((bm, c), lambda i: (i, 0)),
        out_shape=jax.ShapeDtypeStruct((M, c), jnp.float32),
        interpret=_INTERPRET,
    )(gpx, npos, wpad, g[None, :], b[None, :])


def _td_oh_pl(ptabx, kidx, npos, wpad, g, b, ns, c):
    nt, Dp = ptabx.shape
    M = npos.shape[0]

    def body(tab_ref, idx_ref, np_ref, w_ref, g_ref, b_ref, o_ref):
        MN = M * ns
        idx3 = idx_ref[...][:, :, None]
        iota = lax.broadcasted_iota(jnp.int32, (M, ns, nt), 2)
        oh = jnp.where(iota == idx3, 1.0, 0.0).reshape(MN, nt)
        gpx = jnp.dot(oh, tab_ref[...], preferred_element_type=jnp.float32)
        wfull = w_ref[...]
        h = jnp.dot(gpx, wfull, preferred_element_type=jnp.float32)
        corr = jnp.dot(np_ref[...], wfull[:3], preferred_element_type=jnp.float32)
        h3 = h.reshape(M, ns, c) - corr[:, None, :]
        h3 = jnp.maximum(h3 * g_ref[...][None] + b_ref[...][None], 0.0)
        out = h3[:, 0, :]
        for j in range(1, ns):
            out = jnp.maximum(out, h3[:, j, :])
        o_ref[...] = out

    return pl.pallas_call(
        body,
        out_shape=jax.ShapeDtypeStruct((M, c), jnp.float32),
        interpret=_INTERPRET,
    )(ptabx, kidx, npos, wpad, g[None, :], b[None, :])


def _blocks_fused_pl(x, p16, bidx, blks, ns, c):
    M = x.shape[0]
    cs = c // 8
    nb = len(blks)

    def body(*refs):
        x_ref, p16_ref, bidx_ref = refs[0:3]
        prefs = refs[3:-1]
        o_ref = refs[-1]
        MN = M * ns
        idx3 = bidx_ref[...][:, :, None]
        iota = lax.broadcasted_iota(jnp.int32, (M, ns, M), 2)
        oh = jnp.where(iota == idx3, 1.0, 0.0).reshape(MN, M)
        p16full = p16_ref[...]
        pg16 = jnp.dot(oh, p16full, preferred_element_type=jnp.float32)
        xcur = x_ref[...]
        for bi in range(nb):
            (w1, g1, b1, wq, bq, wk, bk, wv, bv,
             wp1, bp1, gp, bpn, wp2, bp2,
             gw1, bw1, ww1, bww1, gw2, bw2, ww2, bww2,
             g2, b2, w3, g3, b3) = [r[...] for r in prefs[bi * 28:(bi + 1) * 28]]
            y = jnp.maximum(
                jnp.dot(xcur, w1, preferred_element_type=jnp.float32) * g1 + b1,
                0.0)
            q = jnp.dot(y, wq, preferred_element_type=jnp.float32) + bq
            kk = jnp.dot(y, wk, preferred_element_type=jnp.float32) + bk
            vv = jnp.dot(y, wv, preferred_element_type=jnp.float32) + bv
            kv = jnp.concatenate([kk, vv], axis=1)
            gkv = jnp.dot(oh, kv, preferred_element_type=jnp.float32)
            a = jnp.dot(pg16, wp1, preferred_element_type=jnp.float32)
            corr = jnp.dot(p16full[:, :3], wp1[:3],
                           preferred_element_type=jnp.float32)
            a3 = a.reshape(M, ns, 3) - corr[:, None, :] + bp1[None]
            a3 = jnp.maximum(a3 * gp[None] + bpn[None], 0.0)
            pr = jnp.dot(a3.reshape(MN, 3), wp2,
                         preferred_element_type=jnp.float32) + bp2
            gk = gkv[:, :c]
            gv = gkv[:, c:]
            t3 = (gk + pr).reshape(M, ns, c) - q[:, None, :]
            w = jnp.maximum(t3 * gw1[None] + bw1[None], 0.0)
            w = jnp.dot(w.reshape(MN, c), ww1,
                        preferred_element_type=jnp.float32) + bww1
            w = jnp.maximum(w * gw2 + bw2, 0.0)
            w = jnp.dot(w, ww2, preferred_element_type=jnp.float32) + bww2
            w3d = w.reshape(M, ns, cs)
            mx = w3d[:, 0:1, :]
            for j in range(1, ns):
                mx = jnp.maximum(mx, w3d[:, j:j + 1, :])
            e = jnp.exp(w3d - mx)
            ssum = e[:, 0:1, :]
            for j in range(1, ns):
                ssum = ssum + e[:, j:j + 1, :]
            sm = e / ssum
            wfull = jnp.concatenate([sm] * 8, axis=2)
            vpr = (gv + pr).reshape(M, ns, c) * wfull
            out = vpr[:, 0, :]
            for j in range(1, ns):
                out = out + vpr[:, j, :]
            z = jnp.maximum(out * g2 + b2, 0.0)
            o = (jnp.dot(z, w3, preferred_element_type=jnp.float32)
                 * g3 + b3 + xcur)
            xcur = jnp.maximum(o, 0.0)
        o_ref[...] = xcur

    ops = [x, p16, bidx]
    for bp in blks:
        ap = bp['attn']
        wp1pad = jnp.zeros((16, 3), jnp.float32).at[:3].set(ap['wp1'])
        ops += [bp['w1'], bp['g1'][None, :], bp['b1'][None, :],
                ap['wq'], ap['bq'][None, :], ap['wk'], ap['bk'][None, :],
                ap['wv'], ap['bv'][None, :],
                wp1pad, ap['bp1'][None, :], ap['gp'][None, :],
                ap['bpn'][None, :], ap['wp2'], ap['bp2'][None, :],
                ap['gw1'][None, :], ap['bw1'][None, :], ap['ww1'],
                ap['bww1'][None, :], ap['gw2'][None, :], ap['bw2'][None, :],
                ap['ww2'], ap['bww2'][None, :],
                bp['g2'][None, :], bp['b2'][None, :], bp['w3'],
                bp['g3'][None, :], bp['b3'][None, :]]
    return pl.pallas_call(
        body,
        out_shape=jax.ShapeDtypeStruct((M, c), jnp.float32),
        interpret=_INTERPRET,
    )(*ops)


def _qkv_pl(x, w1, g1, b1, wq, bq, wk, bk, wv, bv):
    M, c = x.shape
    bm = min(M, 1024)

    def body(x_ref, w1_r, g1_r, b1_r, wq_r, bq_r, wk_r, bk_r, wv_r, bv_r,
             q_ref, kv_ref):
        xb = x_ref[...]
        y = jnp.maximum(
            jnp.dot(xb, w1_r[...], preferred_element_type=jnp.float32)
            * g1_r[...] + b1_r[...], 0.0)
        q_ref[...] = jnp.dot(y, wq_r[...], preferred_element_type=jnp.float32) + bq_r[...]
        kk = jnp.dot(y, wk_r[...], preferred_element_type=jnp.float32) + bk_r[...]
        vv = jnp.dot(y, wv_r[...], preferred_element_type=jnp.float32) + bv_r[...]
        kv_ref[...] = jnp.concatenate([kk, vv], axis=1)

    mat = lambda: pl.BlockSpec((c, c), lambda i: (0, 0))
    vec = lambda: pl.BlockSpec((1, c), lambda i: (0, 0))
    return pl.pallas_call(
        body,
        grid=(M // bm,),
        in_specs=[pl.BlockSpec((bm, c), lambda i: (i, 0)),
                  mat(), vec(), vec(), mat(), vec(), mat(), vec(), mat(), vec()],
        out_specs=[pl.BlockSpec((bm, c), lambda i: (i, 0)),
                   pl.BlockSpec((bm, 2 * c), lambda i: (i, 0))],
        out_shape=[jax.ShapeDtypeStruct((M, c), jnp.float32),
                   jax.ShapeDtypeStruct((M, 2 * c), jnp.float32)],
        interpret=_INTERPRET,
    )(x, w1, g1[None, :], b1[None, :], wq, bq[None, :], wk, bk[None, :],
      wv, bv[None, :])


def _attn_pl(xin, q, gkv, pg, p, ns, c, ap, g2, b2, w3m, g3, b3):
    M = xin.shape[0]
    cs = c // 8
    bm = min(M, 256)
    wp1pad = jnp.zeros((16, 3), jnp.float32).at[:3].set(ap['wp1'])

    def body(xin_ref, q_ref, gkv_ref, pg_ref, p_ref,
             wp1_r, bp1_r, gp_r, bpn_r, wp2_r, bp2_r,
             gw1_r, bw1_r, ww1_r, bww1_r, gw2_r, bw2_r, ww2_r, bww2_r,
             g2_r, b2_r, w3_r, g3_r, b3_r, o_ref):
        BN = bm * ns
        a = jnp.dot(pg_ref[...], wp1_r[...], preferred_element_type=jnp.float32)
        corr = jnp.dot(p_ref[...], wp1_r[...][:3],
                       preferred_element_type=jnp.float32)
        a3 = a.reshape(bm, ns, 3) - corr[:, None, :] + bp1_r[...][None]
        a3 = jnp.maximum(a3 * gp_r[...][None] + bpn_r[...][None], 0.0)
        pr = jnp.dot(a3.reshape(BN, 3), wp2_r[...],
                     preferred_element_type=jnp.float32) + bp2_r[...]
        g = gkv_ref[...]
        gk = g[:, :c]
        gv = g[:, c:]
        t3 = (gk + pr).reshape(bm, ns, c) - q_ref[...][:, None, :]
        w = jnp.maximum(t3 * gw1_r[...][None] + bw1_r[...][None], 0.0)
        w = jnp.dot(w.reshape(BN, c), ww1_r[...],
                    preferred_element_type=jnp.float32) + bww1_r[...]
        w = jnp.maximum(w * gw2_r[...] + bw2_r[...], 0.0)
        w = jnp.dot(w, ww2_r[...], preferred_element_type=jnp.float32) + bww2_r[...]
        w3d = w.reshape(bm, ns, cs)
        mx = w3d[:, 0:1, :]
        for j in range(1, ns):
            mx = jnp.maximum(mx, w3d[:, j:j + 1, :])
        e = jnp.exp(w3d - mx)
        ssum = e[:, 0:1, :]
        for j in range(1, ns):
            ssum = ssum + e[:, j:j + 1, :]
        sm = e / ssum
        wfull = jnp.concatenate([sm] * 8, axis=2)
        vpr = (gv + pr).reshape(bm, ns, c) * wfull
        out = vpr[:, 0, :]
        for j in range(1, ns):
            out = out + vpr[:, j, :]
        z = jnp.maximum(out * g2_r[...] + b2_r[...], 0.0)
        o = (jnp.dot(z, w3_r[...], preferred_element_type=jnp.float32)
             * g3_r[...] + b3_r[...] + xin_ref[...])
        o_ref[...] = jnp.maximum(o, 0.0)

    row = lambda w: pl.BlockSpec((bm, w), lambda i: (i, 0))
    rowns = lambda w: pl.BlockSpec((bm * ns, w), lambda i: (i, 0))
    full = lambda a, bdim: pl.BlockSpec((a, bdim), lambda i: (0, 0))
    cs_ = cs
    return pl.pallas_call(
        body,
        grid=(M // bm,),
        in_specs=[
            row(c), row(c), rowns(2 * c), rowns(16), row(3),
            full(16, 3), full(1, 3), full(1, 3), full(1, 3),
            full(3, c), full(1, c),
            full(1, c), full(1, c), full(c, cs_), full(1, cs_),
            full(1, cs_), full(1, cs_), full(cs_, cs_), full(1, cs_),
            full(1, c), full(1, c), full(c, c), full(1, c), full(1, c),
        ],
        out_specs=row(c),
        out_shape=jax.ShapeDtypeStruct((M, c), jnp.float32),
        interpret=_INTERPRET,
    )(xin, q, gkv, pg, p,
      wp1pad, ap['bp1'][None, :], ap['gp'][None, :], ap['bpn'][None, :],
      ap['wp2'], ap['bp2'][None, :],
      ap['gw1'][None, :], ap['bw1'][None, :], ap['ww1'], ap['bww1'][None, :],
      ap['gw2'][None, :], ap['bw2'][None, :], ap['ww2'], ap['bww2'][None, :],
      g2[None, :], b2[None, :], w3m, g3[None, :], b3[None, :])


def _pT8(p):
    Bb, n, _ = p.shape
    pt = jnp.transpose(p, (0, 2, 1))
    return jnp.concatenate([pt, jnp.zeros((Bb, 5, n), jnp.float32)], axis=1)


def kernel(inputs, params):
    pxo = jnp.transpose(inputs[0], (0, 2, 1))
    p = pxo[:, :, :3]
    x = pxo
    n = _N
    outs = []
    for i in range(5):
        sp = params[i]
        ns = _NSAMPLE[i]
        c = _PLANES[i]
        td = sp['td']
        if _STRIDE[i] == 1:
            x = _lin_relu_pl(x.reshape(_B * n, -1), td['w'], td['g'],
                             td['b']).reshape(_B, n, c)
        else:
            cin = x.shape[-1]
            m = n // _STRIDE[i]
            p4 = jnp.transpose(p, (0, 2, 1)).reshape(_B, 3, 8, n // 8)
            nposT = _fps_pl(p4, m)
            nposb = jnp.transpose(nposT[:, :3, :], (0, 2, 1))
            npos = nposb.reshape(_B * m, 3)
            goff = (jnp.arange(_B, dtype=jnp.int32) * n)[:, None]
            kidx = _knn_pl(nposb, _pT8(p), ns)
            kglob = (kidx + goff[:, :, None]).reshape(_B * m, ns)
            Dp = ((8 + cin + 15) // 16) * 16
            ptabx = _pad_cols(
                jnp.concatenate([_pad_cols(p.reshape(_B * n, 3), 8),
                                 x.reshape(_B * n, cin)], axis=1), Dp)
            wpad = jnp.zeros((Dp, c), jnp.float32)
            wpad = wpad.at[:3].set(td['w'][:3]).at[8:8 + cin].set(td['w'][3:])
            if _B * n <= 512:
                x = _td_oh_pl(ptabx, kglob, npos, wpad, td['g'], td['b'],
                              ns, c)
            else:
                gpx = _sc_gather(ptabx, kglob.reshape(-1))
                x = _td_pl(gpx, npos, wpad, td['g'], td['b'], ns, c)
            x = x.reshape(_B, m, c)
            p = nposb
            n = m
        goff = (jnp.arange(_B, dtype=jnp.int32) * n)[:, None, None]
        bidx = _knn_pl(p, _pT8(p), ns)
        bglob = (bidx + goff).reshape(_B * n, ns)
        if _B * n <= 512:
            p16 = _pad_cols(p.reshape(_B * n, 3), 16)
            xf = _blocks_fused_pl(x.reshape(_B * n, c), p16, bglob,
                                  sp['blocks'], ns, c)
            x = xf.reshape(_B, n, c)
        else:
            bflat = bglob.reshape(-1)
            ptab = _pad_cols(p.reshape(_B * n, 3), 16)
            pg = _sc_gather(ptab, bflat)
            pflat = p.reshape(_B * n, 3)
            for bp in sp['blocks']:
                xf = x.reshape(_B * n, c)
                ap = bp['attn']
                q, kv = _qkv_pl(xf, bp['w1'], bp['g1'], bp['b1'],
                                ap['wq'], ap['bq'], ap['wk'], ap['bk'],
                                ap['wv'], ap['bv'])
                gkv = _sc_gather(kv, bflat)
                xf = _attn_pl(xf, q, gkv, pg, pflat, ns, c, ap,
                              bp['g2'], bp['b2'], bp['w3'], bp['g3'],
                              bp['b3'])
                x = xf.reshape(_B, n, c)
        outs.append((p, x))
    res = []
    for pp, xx in outs:
        res.append(pp.reshape(-1, 3))
        res.append(xx.reshape(-1, xx.shape[-1]))
    return tuple(res)
